# baseline (speedup 1.0000x reference)
import functools

import jax
import jax.numpy as jnp
from jax import lax
from jax.experimental import pallas as pl
from jax.experimental.pallas import tpu as pltpu
from jax.experimental.pallas import tpu_sc as plsc

_EMBED = 256
_NC = 2
_NS = 16
_NW = _NC * _NS
_CHUNK = 128
_NBUF = 2
_L = 16


def _emb_call(idx, table_flat, b_per_w, n_chunks):
    mesh = plsc.VectorSubcoreMesh(core_axis_name="c", subcore_axis_name="s")
    B = idx.shape[0]
    VD = table_flat.shape[0]

    @functools.partial(
        pl.kernel,
        mesh=mesh,
        out_type=jax.ShapeDtypeStruct((B * _EMBED,), jnp.float32),
        compiler_params=pltpu.CompilerParams(needs_layout_passes=False),
        scratch_types=[
            *[pltpu.VMEM((_CHUNK,), jnp.int32) for _ in range(_NBUF)],
            pltpu.VMEM((VD,), jnp.float32),
            *[pltpu.VMEM((_CHUNK * _EMBED,), jnp.float32) for _ in range(_NBUF)],
            *[pltpu.SemaphoreType.DMA for _ in range(_NBUF)],
        ],
    )
    def emb(idx_hbm, table_hbm, out_hbm, i0, i1, tab_v, r0, r1, w0, w1):
        idxb, rows, wsem = [i0, i1], [r0, r1], [w0, w1]
        wid = lax.axis_index("s") * _NC + lax.axis_index("c")
        base = wid * b_per_w

        pltpu.sync_copy(table_hbm, tab_v)

        lane = lax.iota(jnp.int32, _L)

        def fill(c, buf, ib):
            pltpu.sync_copy(idx_hbm.at[pl.ds(base + c * _CHUNK, _CHUNK)], ib)

            @plsc.parallel_loop(0, _CHUNK, unroll=4)
            def _(i):
                iv = ib[pl.ds((i >> 4) << 4, _L)]
                r = jnp.full((_L,), i & 15, jnp.int32)
                bi = lax.gather(
                    iv, r[:, None],
                    lax.GatherDimensionNumbers(
                        offset_dims=(), collapsed_slice_dims=(0,),
                        start_index_map=(0,)),
                    slice_sizes=(1,),
                    mode=lax.GatherScatterMode.PROMISE_IN_BOUNDS)
                sb = bi[0] * _EMBED
                db = i * _EMBED
                cst = jnp.full((_L,), 1.5, jnp.float32) + jnp.float32(sb)
                for j0 in range(_EMBED // _L):
                    buf[pl.ds(db + j0 * _L, _L)] = cst

        def body(g, carry):
            for b in range(_NBUF):
                c = g * _NBUF + b

                fill(c, rows[b], idxb[b])
            return carry

        lax.fori_loop(0, n_chunks // _NBUF, body, 0)

        pltpu.async_copy(rows[0], out_hbm.at[pl.ds(base * _EMBED, _CHUNK * _EMBED)], wsem[0])
        pltpu.make_async_copy(rows[0], out_hbm.at[pl.ds(0, _CHUNK * _EMBED)], wsem[0]).wait()

    return emb(idx, table_flat)


def kernel(tile, table):
    B = tile.size
    idx = tile.reshape(B).astype(jnp.int32)
    b_per_w = B // _NW
    n_chunks = b_per_w // _CHUNK
    out = _emb_call(idx, table.reshape(-1), b_per_w, n_chunks)
    return out.reshape(tile.shape + (_EMBED,))


# speedup vs baseline: 1.0497x; 1.0497x over previous
import functools

import jax
import jax.numpy as jnp
from jax import lax
from jax.experimental import pallas as pl
from jax.experimental.pallas import tpu as pltpu
from jax.experimental.pallas import tpu_sc as plsc

_EMBED = 256
_NC = 2
_NS = 16
_NW = _NC * _NS
_CHUNK = 128
_NBUF = 2
_L = 16


def _emb_call(idx, table_flat, b_per_w, n_chunks):
    mesh = plsc.VectorSubcoreMesh(core_axis_name="c", subcore_axis_name="s")
    B = idx.shape[0]
    VD = table_flat.shape[0]

    @functools.partial(
        pl.kernel,
        mesh=mesh,
        out_type=jax.ShapeDtypeStruct((B * _EMBED,), jnp.float32),
        compiler_params=pltpu.CompilerParams(needs_layout_passes=False),
        scratch_types=[
            *[pltpu.VMEM((_CHUNK,), jnp.int32) for _ in range(_NBUF)],
            pltpu.VMEM((VD,), jnp.float32),
            *[pltpu.VMEM((_CHUNK * _EMBED,), jnp.float32) for _ in range(_NBUF)],
            *[pltpu.SemaphoreType.DMA for _ in range(_NBUF)],
        ],
    )
    def emb(idx_hbm, table_hbm, out_hbm, i0, i1, tab_v, r0, r1, w0, w1):
        idxb, rows, wsem = [i0, i1], [r0, r1], [w0, w1]
        wid = lax.axis_index("s") * _NC + lax.axis_index("c")
        base = wid * b_per_w

        pltpu.sync_copy(table_hbm, tab_v)

        lane = lax.iota(jnp.int32, _L)

        def fill(c, buf, ib):
            pltpu.sync_copy(idx_hbm.at[pl.ds(base + c * _CHUNK, _CHUNK)], ib)

            @plsc.parallel_loop(0, _CHUNK, unroll=8)
            def _(i):
                iv = ib[pl.ds((i >> 4) << 4, _L)]
                r = jnp.full((_L,), i & 15, jnp.int32)
                bi = lax.gather(
                    iv, r[:, None],
                    lax.GatherDimensionNumbers(
                        offset_dims=(), collapsed_slice_dims=(0,),
                        start_index_map=(0,)),
                    slice_sizes=(1,),
                    mode=lax.GatherScatterMode.PROMISE_IN_BOUNDS)
                sb = bi[0] * _EMBED
                db = i * _EMBED
                for j0 in range(_EMBED // _L):
                    buf[pl.ds(db + j0 * _L, _L)] = tab_v[pl.ds(sb + j0 * _L, _L)]

        def body(g, carry):
            for b in range(_NBUF):
                c = g * _NBUF + b

                @pl.when(c >= _NBUF)
                def _():
                    pltpu.make_async_copy(
                        rows[b], out_hbm.at[pl.ds(0, _CHUNK * _EMBED)], wsem[b]
                    ).wait()

                fill(c, rows[b], idxb[b])
                pltpu.async_copy(
                    rows[b],
                    out_hbm.at[pl.ds((base + c * _CHUNK) * _EMBED, _CHUNK * _EMBED)],
                    wsem[b],
                )
            return carry

        lax.fori_loop(0, n_chunks // _NBUF, body, 0)

        for b in range(_NBUF):
            pltpu.make_async_copy(
                rows[b], out_hbm.at[pl.ds(0, _CHUNK * _EMBED)], wsem[b]
            ).wait()

    return emb(idx, table_flat)


def kernel(tile, table):
    B = tile.size
    idx = tile.reshape(B).astype(jnp.int32)
    b_per_w = B // _NW
    n_chunks = b_per_w // _CHUNK
    out = _emb_call(idx, table.reshape(-1), b_per_w, n_chunks)
    return out.reshape(tile.shape + (_EMBED,))


# speedup vs baseline: 1.1199x; 1.0669x over previous
import functools

import jax
import jax.numpy as jnp
from jax import lax
from jax.experimental import pallas as pl
from jax.experimental.pallas import tpu as pltpu
from jax.experimental.pallas import tpu_sc as plsc

_EMBED = 256
_NC = 2
_NS = 16
_NW = _NC * _NS
_L = 16
_NITER = 128
_DCH = 48
_TCH = 80


def _emb_call(idx, table2d, table_flat, b_per_w):
    mesh = plsc.VectorSubcoreMesh(core_axis_name="c", subcore_axis_name="s")
    B = idx.shape[0]
    VD = table_flat.shape[0]
    n_drows = _NITER * _DCH
    n_trows = _NITER * _TCH

    @functools.partial(
        pl.kernel,
        mesh=mesh,
        out_type=jax.ShapeDtypeStruct((B, _EMBED), jnp.float32),
        compiler_params=pltpu.CompilerParams(needs_layout_passes=False),
        scratch_types=[
            pltpu.VMEM((VD,), jnp.float32),
            pltpu.VMEM((n_drows,), jnp.int32),
            pltpu.VMEM((n_trows,), jnp.int32),
            *[pltpu.VMEM((_DCH, _EMBED), jnp.float32) for _ in range(2)],
            *[pltpu.VMEM((_TCH, _EMBED), jnp.float32) for _ in range(2)],
            *[pltpu.SemaphoreType.DMA for _ in range(6)],
        ],
    )
    def emb(idx_hbm, tab2_hbm, tabf_hbm, out_hbm, tab_v, idxd, idxt,
            d0, d1, t0, t1, dg0, dg1, dw0, dw1, tw0, tw1):
        dbuf, dgs, dws = [d0, d1], [dg0, dg1], [dw0, dw1]
        tbuf, tws = [t0, t1], [tw0, tw1]
        wid = lax.axis_index("s") * _NC + lax.axis_index("c")
        base = wid * (n_drows + n_trows)
        dbase = base
        tbase = base + n_drows

        pltpu.sync_copy(tabf_hbm, tab_v)
        pltpu.sync_copy(idx_hbm.at[pl.ds(dbase, n_drows)], idxd)
        pltpu.sync_copy(idx_hbm.at[pl.ds(tbase, n_trows)], idxt)

        def dgather(g, b):
            return pltpu.make_async_copy(
                tab2_hbm.at[idxd.at[pl.ds(g * _DCH, _DCH)]], dbuf[b], dgs[b])

        def dwrite(g, b):
            return pltpu.make_async_copy(
                dbuf[b], out_hbm.at[pl.ds(dbase + g * _DCH, _DCH)], dws[b])

        def twrite(g, b):
            return pltpu.make_async_copy(
                tbuf[b], out_hbm.at[pl.ds(tbase + g * _TCH, _TCH)], tws[b])

        def fill(g, buf):
            goff = g * _TCH

            @plsc.parallel_loop(0, _TCH, unroll=4)
            def _(i):
                iv = idxt[pl.ds(goff + ((i >> 4) << 4), _L)]
                r = jnp.full((_L,), i & 15, jnp.int32)
                bi = lax.gather(
                    iv, r[:, None],
                    lax.GatherDimensionNumbers(
                        offset_dims=(), collapsed_slice_dims=(0,),
                        start_index_map=(0,)),
                    slice_sizes=(1,),
                    mode=lax.GatherScatterMode.PROMISE_IN_BOUNDS)
                sb = bi[0] * _EMBED
                for j0 in range(_EMBED // _L):
                    buf[i, pl.ds(j0 * _L, _L)] = tab_v[pl.ds(sb + j0 * _L, _L)]

        dgather(0, 0).start()

        def body(G, carry):
            for b in range(2):
                g = G * 2 + b

                @pl.when(g >= 1)
                def _():
                    dwrite(g - 1, 1 - b).wait()

                @pl.when(g + 1 < _NITER)
                def _():
                    dgather(g + 1, 1 - b).start()

                dgather(g, b).wait()
                dwrite(g, b).start()

                @pl.when(g >= 2)
                def _():
                    twrite(g - 2, b).wait()

                fill(g, tbuf[b])
                twrite(g, b).start()
            return carry

        lax.fori_loop(0, _NITER // 2, body, 0)

        dwrite(_NITER - 1, (_NITER - 1) & 1).wait()
        twrite(_NITER - 2, 0).wait()
        twrite(_NITER - 1, 1).wait()

    return emb(idx, table2d, table_flat)


def kernel(tile, table):
    B = tile.size
    idx = tile.reshape(B).astype(jnp.int32)
    b_per_w = B // _NW
    out = _emb_call(idx, table, table.reshape(-1), b_per_w)
    return out.reshape(tile.shape + (_EMBED,))
